# baseline (device time: 26525 ns/iter reference)
import jax
import jax.numpy as jnp
from jax import lax
from jax.experimental import pallas as pl
from jax.experimental.pallas import tpu as pltpu

B = 2
SQ = 128
HQ = 4
DH = 64
DMODEL = 512
DQK = HQ * DH
NSLOT = 4
NHOP = 3


def kernel(x, Wq, K_ext, V_ext, Wo):
    def body(x_ref, wq_ref, k_ref, v_ref, wo_ref, out_ref,
             kv_gath, send_sems, recv_sems):
        p = lax.axis_index("i")

        nxt = jnp.where(p < 2, p + 4,
              jnp.where(p < 4, p - 2,
              jnp.where(p < 6, p + 2, p - 4)))
        prv = jnp.where(p < 2, p + 2,
              jnp.where(p < 4, p + 4,
              jnp.where(p < 6, p - 4, p - 2)))
        j = jnp.where(p < 2, 0,
            jnp.where(p < 4, 3,
            jnp.where(p < 6, 1, 2)))

        for b in range(B):
            kv_gath[j, 0, b] = k_ref[b].reshape(SQ, DQK).astype(jnp.bfloat16)
            kv_gath[j, 1, b] = v_ref[b].reshape(SQ, DQK).astype(jnp.bfloat16)

        barrier_sem = pltpu.get_barrier_semaphore()
        for nbr in (nxt, prv):
            pl.semaphore_signal(barrier_sem, inc=1, device_id=(nbr,),
                                device_id_type=pl.DeviceIdType.MESH)
        pl.semaphore_wait(barrier_sem, 2)

        for h in range(NHOP):
            src_slot = (j - h) % NSLOT
            rdma = pltpu.make_async_remote_copy(
                src_ref=kv_gath.at[src_slot],
                dst_ref=kv_gath.at[src_slot],
                send_sem=send_sems.at[h],
                recv_sem=recv_sems.at[h],
                device_id=(nxt,),
                device_id_type=pl.DeviceIdType.MESH,
            )
            rdma.start()
            rdma.wait()

        r = lax.broadcasted_iota(jnp.int32, (SQ, NSLOT * SQ), 0)
        c = lax.broadcasted_iota(jnp.int32, (SQ, NSLOT * SQ), 1)
        mask = (r // 64) == ((c // 64) % 2)

        wq_b = wq_ref[...].astype(jnp.bfloat16)
        wo_b = wo_ref[...].astype(jnp.bfloat16)

        for b in range(B):
            xb = x_ref[b].astype(jnp.bfloat16)
            q = jax.lax.dot_general(
                xb, wq_b, (((1,), (0,)), ((), ())),
                preferred_element_type=jnp.float32)
            q = (q * 0.125).astype(jnp.bfloat16)

            ctx_heads = []
            for h in range(HQ):
                qh = q[:, h * DH:(h + 1) * DH]
                k_cat = jnp.concatenate(
                    [kv_gath[g, 0, b][:, h * DH:(h + 1) * DH]
                     for g in range(NSLOT)], axis=0)
                v_cat = jnp.concatenate(
                    [kv_gath[g, 1, b][:, h * DH:(h + 1) * DH]
                     for g in range(NSLOT)], axis=0)
                s = jax.lax.dot_general(
                    qh, k_cat, (((1,), (1,)), ((), ())),
                    preferred_element_type=jnp.float32)
                s = jnp.where(mask, s, -1e9)
                m = jnp.max(s, axis=1, keepdims=True)
                e = jnp.exp(s - m)
                w = e / jnp.sum(e, axis=1, keepdims=True)
                ctx_heads.append(jax.lax.dot_general(
                    w.astype(jnp.bfloat16), v_cat, (((1,), (0,)), ((), ())),
                    preferred_element_type=jnp.float32))

            ctx = jnp.concatenate(ctx_heads, axis=1).astype(jnp.bfloat16)
            out_ref[b] = jax.lax.dot_general(
                ctx, wo_b, (((1,), (0,)), ((), ())),
                preferred_element_type=jnp.float32)

    return pl.pallas_call(
        body,
        out_shape=jax.ShapeDtypeStruct((B, SQ, DMODEL), jnp.float32),
        in_specs=[pl.BlockSpec(memory_space=pltpu.VMEM)] * 5,
        out_specs=pl.BlockSpec(memory_space=pltpu.VMEM),
        scratch_shapes=[
            pltpu.VMEM((NSLOT, 2, B, SQ, DQK), jnp.bfloat16),
            pltpu.SemaphoreType.DMA((NHOP,)),
            pltpu.SemaphoreType.DMA((NHOP,)),
        ],
        compiler_params=pltpu.CompilerParams(collective_id=0),
    )(x, Wq, K_ext, V_ext, Wo)


# device time: 18185 ns/iter; 1.4586x vs baseline; 1.4586x over previous
import jax
import jax.numpy as jnp
from jax import lax
from jax.experimental import pallas as pl
from jax.experimental.pallas import tpu as pltpu

B = 2
SQ = 128
HQ = 4
DH = 64
DMODEL = 512
DQK = HQ * DH
NSLOT = 4
NHOP = 3


def kernel(x, Wq, K_ext, V_ext, Wo):
    def body(x_ref, wq_ref, k_ref, v_ref, wo_ref, out_ref,
             kv_gath, send_sems, recv_sems):
        p = lax.axis_index("i")

        nxt = jnp.where(p < 2, p + 4,
              jnp.where(p < 4, p - 2,
              jnp.where(p < 6, p + 2, p - 4)))
        prv = jnp.where(p < 2, p + 2,
              jnp.where(p < 4, p + 4,
              jnp.where(p < 6, p - 4, p - 2)))
        opp = jnp.where(p < 2, p + 6,
              jnp.where(p < 4, p + 2,
              jnp.where(p < 6, p - 2, p - 6)))

        for b in range(B):
            kv_gath[0, 0, b] = k_ref[b].reshape(SQ, DQK).astype(jnp.bfloat16)
            kv_gath[0, 1, b] = v_ref[b].reshape(SQ, DQK).astype(jnp.bfloat16)

        barrier_sem = pltpu.get_barrier_semaphore()
        for nbr in (nxt, opp, prv):
            pl.semaphore_signal(barrier_sem, inc=1, device_id=(nbr,),
                                device_id_type=pl.DeviceIdType.MESH)
        pl.semaphore_wait(barrier_sem, 3)

        rdmas = []
        for o, tgt in ((2, opp), (1, nxt), (3, prv)):
            r = pltpu.make_async_remote_copy(
                src_ref=kv_gath.at[0],
                dst_ref=kv_gath.at[o],
                send_sem=send_sems.at[o - 1],
                recv_sem=recv_sems.at[o - 1],
                device_id=(tgt,),
                device_id_type=pl.DeviceIdType.MESH,
            )
            r.start()
            rdmas.append((o, r))

        r_ = lax.broadcasted_iota(jnp.int32, (SQ, SQ), 0)
        c_ = lax.broadcasted_iota(jnp.int32, (SQ, SQ), 1)
        slot_mask = (r_ // 64) == (c_ // 64)

        wq_b = wq_ref[...].astype(jnp.bfloat16)
        wo_b = wo_ref[...].astype(jnp.bfloat16)

        qs = []
        for b in range(B):
            xb = x_ref[b].astype(jnp.bfloat16)
            q = jax.lax.dot_general(
                xb, wq_b, (((1,), (0,)), ((), ())),
                preferred_element_type=jnp.float32)
            qs.append((q * 0.125).astype(jnp.bfloat16))

        def slot_scores(b, h, s):
            qh = qs[b][:, h * DH:(h + 1) * DH]
            k_s = kv_gath[s, 0, b][:, h * DH:(h + 1) * DH]
            sc = jax.lax.dot_general(
                qh, k_s, (((1,), (1,)), ((), ())),
                preferred_element_type=jnp.float32)
            return jnp.where(slot_mask, sc, -1e9)

        scores = [[[None] * NSLOT for _ in range(HQ)] for _ in range(B)]
        for b in range(B):
            for h in range(HQ):
                scores[b][h][0] = slot_scores(b, h, 0)

        for o, r in rdmas:
            r.wait_recv()
            for b in range(B):
                for h in range(HQ):
                    scores[b][h][o] = slot_scores(b, h, o)

        for b in range(B):
            ctx_heads = []
            for h in range(HQ):
                s = jnp.concatenate(scores[b][h], axis=1)
                m = jnp.max(s, axis=1, keepdims=True)
                e = jnp.exp(s - m)
                w = (e / jnp.sum(e, axis=1, keepdims=True)).astype(jnp.bfloat16)
                v_cat = jnp.concatenate(
                    [kv_gath[g, 1, b][:, h * DH:(h + 1) * DH]
                     for g in range(NSLOT)], axis=0)
                ctx_heads.append(jax.lax.dot_general(
                    w, v_cat, (((1,), (0,)), ((), ())),
                    preferred_element_type=jnp.float32))

            ctx = jnp.concatenate(ctx_heads, axis=1).astype(jnp.bfloat16)
            out_ref[b] = jax.lax.dot_general(
                ctx, wo_b, (((1,), (0,)), ((), ())),
                preferred_element_type=jnp.float32)

        for _, r in rdmas:
            r.wait_send()

    return pl.pallas_call(
        body,
        out_shape=jax.ShapeDtypeStruct((B, SQ, DMODEL), jnp.float32),
        in_specs=[pl.BlockSpec(memory_space=pltpu.VMEM)] * 5,
        out_specs=pl.BlockSpec(memory_space=pltpu.VMEM),
        scratch_shapes=[
            pltpu.VMEM((NSLOT, 2, B, SQ, DQK), jnp.bfloat16),
            pltpu.SemaphoreType.DMA((NHOP,)),
            pltpu.SemaphoreType.DMA((NHOP,)),
        ],
        compiler_params=pltpu.CompilerParams(collective_id=0),
    )(x, Wq, K_ext, V_ext, Wo)


# device time: 16353 ns/iter; 1.6220x vs baseline; 1.1120x over previous
import os

import jax
import jax.numpy as jnp
from jax import lax
from jax.experimental import pallas as pl
from jax.experimental.pallas import tpu as pltpu

B = 2
SQ = 128
HQ = 4
DH = 64
DMODEL = 512
DQK = HQ * DH
NSLOT = 4
NHOP = 3

_COMPUTE_ONLY = os.environ.get("KERNEL_COMPUTE_ONLY") == "1"

_SCOPES = os.environ.get("KERNEL_SCOPES") == "1"

_K_ONLY = os.environ.get("KERNEL_K_ONLY") == "1"

import contextlib


def _scope(name):
    return jax.named_scope(name) if _SCOPES else contextlib.nullcontext()


def kernel(x, Wq, K_ext, V_ext, Wo):
    def body(x_ref, wq_ref, k_ref, v_ref, wo_ref, out_ref,
             kv_gath, send_sems, recv_sems):
        p = lax.axis_index("i")

        nxt = jnp.where(p < 2, p + 4,
              jnp.where(p < 4, p - 2,
              jnp.where(p < 6, p + 2, p - 4)))
        prv = jnp.where(p < 2, p + 2,
              jnp.where(p < 4, p + 4,
              jnp.where(p < 6, p - 4, p - 2)))
        opp = jnp.where(p < 2, p + 6,
              jnp.where(p < 4, p + 2,
              jnp.where(p < 6, p - 2, p - 6)))

        if not _COMPUTE_ONLY:
            barrier_sem = pltpu.get_barrier_semaphore()
            for nbr in (nxt, opp, prv):
                pl.semaphore_signal(barrier_sem, inc=1, device_id=(nbr,),
                                    device_id_type=pl.DeviceIdType.MESH)

        with _scope("ph_stage"):
            for b in range(B):
                kv_gath[0, 0, b] = k_ref[b].reshape(SQ, DQK).astype(jnp.bfloat16)
                kv_gath[0, 1, b] = v_ref[b].reshape(SQ, DQK).astype(jnp.bfloat16)

        k_rdmas, v_rdmas = [], []
        if not _COMPUTE_ONLY:
            with _scope("ph_barrier"):
                pl.semaphore_wait(barrier_sem, 3)

            for o, tgt in ((2, opp), (1, nxt), (3, prv)):
                r = pltpu.make_async_remote_copy(
                    src_ref=kv_gath.at[0, 0],
                    dst_ref=kv_gath.at[o, 0],
                    send_sem=send_sems.at[o - 1],
                    recv_sem=recv_sems.at[o - 1],
                    device_id=(tgt,),
                    device_id_type=pl.DeviceIdType.MESH,
                )
                r.start()
                k_rdmas.append((o, r))
            if not _K_ONLY:
                for o, tgt in ((2, opp), (1, nxt), (3, prv)):
                    r = pltpu.make_async_remote_copy(
                        src_ref=kv_gath.at[0, 1],
                        dst_ref=kv_gath.at[o, 1],
                        send_sem=send_sems.at[3 + o - 1],
                        recv_sem=recv_sems.at[3 + o - 1],
                        device_id=(tgt,),
                        device_id_type=pl.DeviceIdType.MESH,
                    )
                    r.start()
                    v_rdmas.append((o, r))

        r_ = lax.broadcasted_iota(jnp.int32, (SQ, SQ), 0)
        c_ = lax.broadcasted_iota(jnp.int32, (SQ, SQ), 1)
        slot_mask = (r_ // 64) == (c_ // 64)

        wq_b = wq_ref[...].astype(jnp.bfloat16)
        wo_b = wo_ref[...].astype(jnp.bfloat16)

        qs = []
        with _scope("ph_qproj"):
            for b in range(B):
                xb = x_ref[b].astype(jnp.bfloat16)
                q = jax.lax.dot_general(
                    xb, wq_b, (((1,), (0,)), ((), ())),
                    preferred_element_type=jnp.float32)
                qs.append((q * 0.125).astype(jnp.bfloat16))

        e_bf = [[[None] * NSLOT for _ in range(HQ)] for _ in range(B)]
        denom = [[None] * HQ for _ in range(B)]
        ctx_acc = [[None] * HQ for _ in range(B)]

        def proc_k(b, h, s):
            qh = qs[b][:, h * DH:(h + 1) * DH]
            k_s = kv_gath[s, 0, b][:, h * DH:(h + 1) * DH]
            sc = jax.lax.dot_general(
                qh, k_s, (((1,), (1,)), ((), ())),
                preferred_element_type=jnp.float32)
            e = jnp.exp(jnp.where(slot_mask, sc, -1e9))
            e_bf[b][h][s] = e.astype(jnp.bfloat16)
            d = jnp.sum(e, axis=1, keepdims=True)
            denom[b][h] = d if denom[b][h] is None else denom[b][h] + d

        def proc_v(b, h, s):
            v_s = kv_gath[s, 1, b][:, h * DH:(h + 1) * DH]
            c = jax.lax.dot_general(
                e_bf[b][h][s], v_s, (((1,), (0,)), ((), ())),
                preferred_element_type=jnp.float32)
            ctx_acc[b][h] = c if ctx_acc[b][h] is None else ctx_acc[b][h] + c

        with _scope("ph_local"):
            for b in range(B):
                for h in range(HQ):
                    proc_k(b, h, 0)
                    proc_v(b, h, 0)

        k_rdmas.sort(key=lambda t: {1: 0, 3: 1, 2: 2}[t[0]])
        v_rdmas.sort(key=lambda t: {1: 0, 3: 1, 2: 2}[t[0]])
        for o, r in k_rdmas:
            with _scope(f"ph_kwait{o}"):
                r.wait_recv()
            with _scope(f"ph_kproc{o}"):
                for b in range(B):
                    for h in range(HQ):
                        proc_k(b, h, o)
        for o, r in v_rdmas:
            with _scope(f"ph_vwait{o}"):
                r.wait_recv()
            with _scope(f"ph_vproc{o}"):
                for b in range(B):
                    for h in range(HQ):
                        proc_v(b, h, o)
        if _COMPUTE_ONLY:
            for o in (1, 2, 3):
                for b in range(B):
                    for h in range(HQ):
                        proc_k(b, h, 0)
                        proc_v(b, h, 0)
        if _K_ONLY:
            for o in (1, 2, 3):
                for b in range(B):
                    for h in range(HQ):
                        proc_v(b, h, o)

        with _scope("ph_tail"):
            for b in range(B):
                ctx = jnp.concatenate(
                    [ctx_acc[b][h] / denom[b][h] for h in range(HQ)],
                    axis=1).astype(jnp.bfloat16)
                out_ref[b] = jax.lax.dot_general(
                    ctx, wo_b, (((1,), (0,)), ((), ())),
                    preferred_element_type=jnp.float32)

        with _scope("ph_drain"):
            for _, r in k_rdmas + v_rdmas:
                r.wait_send()

    return pl.pallas_call(
        body,
        out_shape=jax.ShapeDtypeStruct((B, SQ, DMODEL), jnp.float32),
        in_specs=[pl.BlockSpec(memory_space=pltpu.VMEM)] * 5,
        out_specs=pl.BlockSpec(memory_space=pltpu.VMEM),
        scratch_shapes=[
            pltpu.VMEM((NSLOT, 2, B, SQ, DQK), jnp.bfloat16),
            pltpu.SemaphoreType.DMA((2 * NHOP,)),
            pltpu.SemaphoreType.DMA((2 * NHOP,)),
        ],
        compiler_params=pltpu.CompilerParams(
            collective_id=None if _COMPUTE_ONLY else 0),
    )(x, Wq, K_ext, V_ext, Wo)
